# baseline (speedup 1.0000x reference)
import functools

import jax
import jax.numpy as jnp
from jax import lax
from jax.experimental import pallas as pl
from jax.experimental.pallas import tpu as pltpu
from jax.experimental.pallas import tpu_sc as plsc

_VOCAB = 1_000_000
_HID = 64
_NCLS = 4
_GD = 8
_B = 16384
_L = 200
_NTOK = _B * _L

_NC, _NS = 2, 16
_NW = _NC * _NS

_VBLK = 8192

_GCH = 2048
_PER_W = _NTOK // _NW
_GNCH = _PER_W // _GCH


def _proj_body(tab_ref, wt_ref, b_ref, out_ref):
    out_ref[...] = lax.dot_general(
        tab_ref[...], wt_ref[...],
        dimension_numbers=(((1,), (0,)), ((), ())),
        preferred_element_type=jnp.float32,
        precision=lax.Precision.HIGHEST,
    ) + b_ref[...]


def _project_table(table, w_t, b_row):
    grid = pl.cdiv(_VOCAB, _VBLK)
    return pl.pallas_call(
        _proj_body,
        grid=(grid,),
        in_specs=[
            pl.BlockSpec((_VBLK, _HID), lambda i: (i, 0)),
            pl.BlockSpec((_HID, _GD), lambda i: (0, 0)),
            pl.BlockSpec((1, _GD), lambda i: (0, 0)),
        ],
        out_specs=pl.BlockSpec((_VBLK, _GD), lambda i: (i, 0)),
        out_shape=jax.ShapeDtypeStruct((_VOCAB, _GD), jnp.float32),
    )(table, w_t, b_row)


@functools.lru_cache(maxsize=1)
def _make_gather():
    mesh = plsc.VectorSubcoreMesh(core_axis_name="c", subcore_axis_name="s")

    @functools.partial(
        pl.kernel,
        mesh=mesh,
        out_type=jax.ShapeDtypeStruct((_NTOK, _NCLS), jnp.float32),
        scratch_types=[
            pltpu.VMEM((_GCH,), jnp.int32),
            pltpu.VMEM((_GCH, _GD), jnp.float32),
            pltpu.SemaphoreType.DMA,
        ],
        compiler_params=pltpu.CompilerParams(use_tc_tiling_on_sc=False,
                                             needs_layout_passes=False),
    )
    def gather_kernel(idx_hbm, proj_hbm, out_hbm, idx_v, rows_v, sem):
        wid = lax.axis_index("s") * _NC + lax.axis_index("c")
        kbase = wid * _PER_W

        def chunk_body(i, carry):
            koff = kbase + i * _GCH
            pltpu.sync_copy(idx_hbm.at[pl.ds(koff, _GCH)], idx_v)
            pltpu.async_copy(proj_hbm.at[idx_v], rows_v, sem).wait()
            pltpu.sync_copy(rows_v.at[:, pl.ds(0, _NCLS)],
                            out_hbm.at[pl.ds(koff, _GCH)])
            return carry

        lax.fori_loop(0, _GNCH, chunk_body, 0)

    return gather_kernel


def kernel(inputs, table, W, b):
    w_t = jnp.zeros((_HID, _GD), jnp.float32).at[:, :_NCLS].set(W.T)
    b_row = jnp.zeros((1, _GD), jnp.float32).at[0, :_NCLS].set(b)

    proj = _project_table(table, w_t, b_row)

    flat_idx = inputs.reshape(-1).astype(jnp.int32)
    rows = _make_gather()(flat_idx, proj)

    return rows.reshape(_B, _L, _NCLS)


# speedup vs baseline: 2.9727x; 2.9727x over previous
import functools

import jax
import jax.numpy as jnp
from jax import lax
from jax.experimental import pallas as pl
from jax.experimental.pallas import tpu as pltpu
from jax.experimental.pallas import tpu_sc as plsc

_VOCAB = 1_000_000
_HID = 64
_NCLS = 4
_GD = 8
_B = 16384
_L = 200
_NTOK = _B * _L

_NC, _NS = 2, 16
_NW = _NC * _NS

_VBLK = 8192

_GCH = 2048
_PER_W = _NTOK // _NW
_GNCH = _PER_W // _GCH


def _proj_body(tab_ref, wt_ref, b_ref, out_ref):
    out_ref[...] = lax.dot_general(
        tab_ref[...], wt_ref[...],
        dimension_numbers=(((1,), (0,)), ((), ())),
        preferred_element_type=jnp.float32,
        precision=lax.Precision.HIGHEST,
    ) + b_ref[...]


def _project_table(table, w_t, b_row):
    grid = pl.cdiv(_VOCAB, _VBLK)
    return pl.pallas_call(
        _proj_body,
        grid=(grid,),
        in_specs=[
            pl.BlockSpec((_VBLK, _HID), lambda i: (i, 0)),
            pl.BlockSpec((_HID, _GD), lambda i: (0, 0)),
            pl.BlockSpec((1, _GD), lambda i: (0, 0)),
        ],
        out_specs=pl.BlockSpec((_VBLK, _GD), lambda i: (i, 0)),
        out_shape=jax.ShapeDtypeStruct((_VOCAB, _GD), jnp.float32),
    )(table, w_t, b_row)


@functools.lru_cache(maxsize=1)
def _make_gather():
    mesh = plsc.VectorSubcoreMesh(core_axis_name="c", subcore_axis_name="s")

    @functools.partial(
        pl.kernel,
        mesh=mesh,
        out_type=jax.ShapeDtypeStruct((_NTOK, _GD), jnp.float32),
        scratch_types=[
            pltpu.VMEM((_GCH,), jnp.int32),
            pltpu.VMEM((_GCH, _GD), jnp.float32),
            pltpu.SemaphoreType.DMA,
        ],
        compiler_params=pltpu.CompilerParams(use_tc_tiling_on_sc=False,
                                             needs_layout_passes=False),
    )
    def gather_kernel(idx_hbm, proj_hbm, out_hbm, idx_v, rows_v, sem):
        wid = lax.axis_index("s") * _NC + lax.axis_index("c")
        kbase = wid * _PER_W

        def chunk_body(i, carry):
            koff = kbase + i * _GCH
            pltpu.sync_copy(idx_hbm.at[pl.ds(koff, _GCH)], idx_v)
            pltpu.async_copy(proj_hbm.at[idx_v], rows_v, sem).wait()
            pltpu.sync_copy(rows_v, out_hbm.at[pl.ds(koff, _GCH)])
            return carry

        lax.fori_loop(0, _GNCH, chunk_body, 0)

    return gather_kernel


def kernel(inputs, table, W, b):
    w_t = jnp.zeros((_HID, _GD), jnp.float32).at[:, :_NCLS].set(W.T)
    b_row = jnp.zeros((1, _GD), jnp.float32).at[0, :_NCLS].set(b)

    proj = _project_table(table, w_t, b_row)

    flat_idx = inputs.reshape(-1).astype(jnp.int32)
    rows = _make_gather()(flat_idx, proj)

    return rows.reshape(_B, _L, _GD)[:, :, :_NCLS]


# speedup vs baseline: 3.3999x; 1.1437x over previous
import functools

import jax
import jax.numpy as jnp
from jax import lax
from jax.experimental import pallas as pl
from jax.experimental.pallas import tpu as pltpu
from jax.experimental.pallas import tpu_sc as plsc

_VOCAB = 1_000_000
_HID = 64
_NCLS = 4
_GD = 8
_B = 16384
_L = 200
_NTOK = _B * _L

_NC, _NS = 2, 16
_NW = _NC * _NS

_VBLK = 8192

_GCH = 2048
_PER_W = _NTOK // _NW
_GNCH = _PER_W // _GCH


def _proj_body(tab_ref, wexp_ref, b_ref, out_ref):
    out_ref[...] = lax.dot_general(
        tab_ref[...], wexp_ref[...],
        dimension_numbers=(((1,), (0,)), ((), ())),
        preferred_element_type=jnp.float32,
    ) + b_ref[...]


_PBLK = 512


def _project_table(table16, wexp, b128):
    grid = pl.cdiv(_VOCAB // 16, _PBLK)
    return pl.pallas_call(
        _proj_body,
        grid=(grid,),
        in_specs=[
            pl.BlockSpec((_PBLK, 16 * _HID), lambda i: (i, 0)),
            pl.BlockSpec((16 * _HID, 128), lambda i: (0, 0)),
            pl.BlockSpec((1, 128), lambda i: (0, 0)),
        ],
        out_specs=pl.BlockSpec((_PBLK, 128), lambda i: (i, 0)),
        out_shape=jax.ShapeDtypeStruct((_VOCAB // 16, 128), jnp.float32),
    )(table16, wexp, b128)


@functools.lru_cache(maxsize=1)
def _make_gather():
    mesh = plsc.VectorSubcoreMesh(core_axis_name="c", subcore_axis_name="s")

    @functools.partial(
        pl.kernel,
        mesh=mesh,
        out_type=jax.ShapeDtypeStruct((_NTOK, _GD), jnp.float32),
        scratch_types=[
            pltpu.VMEM((_GCH,), jnp.int32),
            pltpu.VMEM((_GCH, _GD), jnp.float32),
            pltpu.SemaphoreType.DMA,
        ],
        compiler_params=pltpu.CompilerParams(use_tc_tiling_on_sc=False,
                                             needs_layout_passes=False),
    )
    def gather_kernel(idx_hbm, proj_hbm, out_hbm, idx_v, rows_v, sem):
        wid = lax.axis_index("s") * _NC + lax.axis_index("c")
        kbase = wid * _PER_W

        def chunk_body(i, carry):
            koff = kbase + i * _GCH
            pltpu.sync_copy(idx_hbm.at[pl.ds(koff, _GCH)], idx_v)
            pltpu.async_copy(proj_hbm.at[idx_v], rows_v, sem).wait()
            pltpu.sync_copy(rows_v, out_hbm.at[pl.ds(koff, _GCH)])
            return carry

        lax.fori_loop(0, _GNCH, chunk_body, 0)

    return gather_kernel


def kernel(inputs, table, W, b):
    w_pad = jnp.zeros((_GD, _HID), jnp.float32).at[:_NCLS].set(W)
    wexp = jnp.einsum('rR,ch->rhRc', jnp.eye(16, dtype=jnp.float32),
                      w_pad).reshape(16 * _HID, 16 * _GD)
    b_pad = jnp.zeros((_GD,), jnp.float32).at[:_NCLS].set(b)
    b128 = jnp.tile(b_pad, 16).reshape(1, 128)

    table16 = table.reshape(_VOCAB // 16, 16 * _HID)
    proj = _project_table(table16, wexp, b128).reshape(_VOCAB, _GD)

    flat_idx = inputs.reshape(-1).astype(jnp.int32)
    rows = _make_gather()(flat_idx, proj)

    return rows.reshape(_B, _L, _GD)[:, :, :_NCLS]
